# Initial kernel scaffold; baseline (speedup 1.0000x reference)
import jax
import jax.numpy as jnp
from jax.experimental import pallas as pl


def kernel(node_feat, edge_feat, node_label, edge_label, node_batch, edge_batch, Wn1, bn1, Wn2, bn2, We1, be1, We2, be2):
    raise NotImplementedError("write your pallas kernel here")


# speedup vs baseline: 1.9468x; 1.9468x over previous
import jax
import jax.numpy as jnp
from jax.experimental import pallas as pl
from jax.experimental.pallas import tpu as pltpu

_N, _E, _D, _DE = 10000, 320000, 128, 16
_PACK = _D // _DE
_EP = _E // _PACK
_G = 25
_NBLK = _N // _G
_EBLK = _EP // _G


def _bce(logits, labels):
    return (jnp.maximum(logits, 0.0) - logits * labels
            + jnp.log1p(jnp.exp(-jnp.abs(logits))))


def _fused(nf_ref, nlab_ref, ef_ref, elab_ref,
           wn1_ref, bn1_ref, wn2_ref, bn2_ref,
           we1_ref, be1_ref, we2_ref, be2_ref,
           dn_ref, de_ref,
           nlog_ref, elog_ref, nsum_ref, esum_ref):
    i = pl.program_id(0)

    nh = jnp.maximum(
        jnp.dot(nf_ref[...], wn1_ref[...], preferred_element_type=jnp.float32)
        + bn1_ref[...], 0.0)
    nlogit = (jnp.dot(nh, wn2_ref[...], preferred_element_type=jnp.float32)
              + bn2_ref[...])
    nlog_ref[...] = nlogit

    eh = jnp.maximum(
        jnp.dot(ef_ref[...], we1_ref[...], preferred_element_type=jnp.float32)
        + be1_ref[...], 0.0)
    elogit = (jnp.dot(eh, we2_ref[...], preferred_element_type=jnp.float32)
              + be2_ref[...])
    elog_ref[...] = elogit

    @pl.when(i == 0)
    def _init():
        nsum_ref[...] = jnp.zeros_like(nsum_ref)
        esum_ref[...] = jnp.zeros_like(esum_ref)

    nsum_ref[...] += jnp.sum(_bce(nlogit, nlab_ref[...])).reshape(1, 1)
    esum_ref[...] += jnp.sum(_bce(elogit, elab_ref[...])).reshape(1, 1)

    @pl.when(i == _G - 1)
    def _norm():
        nsum_ref[...] = nsum_ref[...] / dn_ref[...]
        esum_ref[...] = esum_ref[...] / de_ref[...]


def kernel(node_feat, edge_feat, node_label, edge_label, node_batch,
           edge_batch, Wn1, bn1, Wn2, bn2, We1, be1, We2, be2):
    ef = edge_feat.reshape(_EP, _D)
    elab = edge_label.reshape(_EP, _PACK)
    nlab = node_label.reshape(_N, 1)

    eye = jnp.eye(_PACK, dtype=We1.dtype)
    We1t = jnp.kron(eye, We1)
    be1t = jnp.tile(be1, _PACK).reshape(1, _D)
    We2t = jnp.kron(eye, We2)

    dn = (node_batch[-1].astype(jnp.float32) + 1.0).reshape(1, 1)
    de = (edge_batch[-1].astype(jnp.float32) + 1.0).reshape(1, 1)

    row = lambda i: (i, 0)
    fixed = lambda i: (0, 0)
    full = lambda a: pl.BlockSpec(a.shape, fixed)

    nlog, elog, nsum, esum = pl.pallas_call(
        _fused,
        grid=(_G,),
        in_specs=[
            pl.BlockSpec((_NBLK, _D), row),
            pl.BlockSpec((_NBLK, 1), row),
            pl.BlockSpec((_EBLK, _D), row),
            pl.BlockSpec((_EBLK, _PACK), row),
            full(Wn1),
            pl.BlockSpec((1, _D), fixed),
            full(Wn2),
            pl.BlockSpec((1, 1), fixed),
            pl.BlockSpec((_D, _D), fixed),
            pl.BlockSpec((1, _D), fixed),
            pl.BlockSpec((_D, _PACK), fixed),
            pl.BlockSpec((1, 1), fixed),
            pl.BlockSpec((1, 1), fixed),
            pl.BlockSpec((1, 1), fixed),
        ],
        out_specs=[
            pl.BlockSpec((_NBLK, 1), row),
            pl.BlockSpec((_EBLK, _PACK), row),
            pl.BlockSpec((1, 1), fixed),
            pl.BlockSpec((1, 1), fixed),
        ],
        out_shape=[
            jax.ShapeDtypeStruct((_N, 1), jnp.float32),
            jax.ShapeDtypeStruct((_EP, _PACK), jnp.float32),
            jax.ShapeDtypeStruct((1, 1), jnp.float32),
            jax.ShapeDtypeStruct((1, 1), jnp.float32),
        ],
        compiler_params=pltpu.CompilerParams(
            dimension_semantics=("arbitrary",)),
    )(node_feat, nlab, ef, elab,
      Wn1, bn1.reshape(1, _D), Wn2, bn2.reshape(1, 1),
      We1t, be1t, We2t, be2.reshape(1, 1),
      dn, de)

    return (nlog.reshape(_N), elog.reshape(_E), nsum[0, 0], esum[0, 0])


# speedup vs baseline: 2.0243x; 1.0398x over previous
import jax
import jax.numpy as jnp
from jax import lax
from jax.experimental import pallas as pl
from jax.experimental.pallas import tpu as pltpu

_N, _E, _D, _DE = 10000, 320000, 128, 16
_PACK = _D // _DE
_EP = _E // _PACK
_G = 25
_NBLK = _N // _G
_EBLK = _EP // _G


def _bce(logits, labels):
    return (jnp.maximum(logits, 0.0) - logits * labels
            + jnp.log1p(jnp.exp(-jnp.abs(logits))))


def _fused(nf_ref, nlab_ref, ef_ref, elab_ref,
           wn1_ref, bn1_ref, wn2_ref, bn2_ref,
           we1_ref, be1_ref, we2_ref, be2_ref,
           dn_ref, de_ref,
           nlog_ref, elog_ref, nsum_ref, esum_ref):
    i = pl.program_id(0)

    nh = jnp.maximum(
        jnp.dot(nf_ref[...], wn1_ref[...], preferred_element_type=jnp.float32)
        + bn1_ref[...], 0.0)
    nlogit = (lax.dot_general(wn2_ref[...], nh, (((0,), (1,)), ((), ())),
                              preferred_element_type=jnp.float32)
              + bn2_ref[...])
    nlog_ref[...] = nlogit.reshape(1, 1, _NBLK)

    eh = jnp.maximum(
        jnp.dot(ef_ref[...], we1_ref[...], preferred_element_type=jnp.float32)
        + be1_ref[...], 0.0)
    elogit = (jnp.dot(eh, we2_ref[...], preferred_element_type=jnp.float32)
              + be2_ref[...])
    elog_ref[...] = elogit

    @pl.when(i == 0)
    def _init():
        nsum_ref[...] = jnp.zeros_like(nsum_ref)
        esum_ref[...] = jnp.zeros_like(esum_ref)

    nlab = nlab_ref[...].reshape(1, _NBLK)
    nsum_ref[...] += jnp.sum(_bce(nlogit, nlab)).reshape(1, 1)
    esum_ref[...] += jnp.sum(_bce(elogit, elab_ref[...])).reshape(1, 1)

    @pl.when(i == _G - 1)
    def _norm():
        nsum_ref[...] = nsum_ref[...] / dn_ref[...]
        esum_ref[...] = esum_ref[...] / de_ref[...]


def kernel(node_feat, edge_feat, node_label, edge_label, node_batch,
           edge_batch, Wn1, bn1, Wn2, bn2, We1, be1, We2, be2):
    ef = edge_feat.reshape(_EP, _D)
    elab = edge_label.reshape(_EP, _PACK)
    nlab = node_label.reshape(_G, 1, _NBLK)

    eye = jnp.eye(_PACK, dtype=We1.dtype)
    We1t = jnp.kron(eye, We1)
    be1t = jnp.tile(be1, _PACK).reshape(1, _D)
    We2t = jnp.kron(eye, We2)

    dn = (node_batch[-1].astype(jnp.float32) + 1.0).reshape(1, 1)
    de = (edge_batch[-1].astype(jnp.float32) + 1.0).reshape(1, 1)

    row = lambda i: (i, 0)
    row3 = lambda i: (i, 0, 0)
    fixed = lambda i: (0, 0)
    full = lambda a: pl.BlockSpec(a.shape, fixed)

    nlog, elog, nsum, esum = pl.pallas_call(
        _fused,
        grid=(_G,),
        in_specs=[
            pl.BlockSpec((_NBLK, _D), row),
            pl.BlockSpec((1, 1, _NBLK), row3),
            pl.BlockSpec((_EBLK, _D), row),
            pl.BlockSpec((_EBLK, _PACK), row),
            full(Wn1),
            pl.BlockSpec((1, _D), fixed),
            full(Wn2),
            pl.BlockSpec((1, 1), fixed),
            pl.BlockSpec((_D, _D), fixed),
            pl.BlockSpec((1, _D), fixed),
            pl.BlockSpec((_D, _PACK), fixed),
            pl.BlockSpec((1, 1), fixed),
            pl.BlockSpec((1, 1), fixed),
            pl.BlockSpec((1, 1), fixed),
        ],
        out_specs=[
            pl.BlockSpec((1, 1, _NBLK), row3),
            pl.BlockSpec((_EBLK, _PACK), row),
            pl.BlockSpec((1, 1), fixed),
            pl.BlockSpec((1, 1), fixed),
        ],
        out_shape=[
            jax.ShapeDtypeStruct((_G, 1, _NBLK), jnp.float32),
            jax.ShapeDtypeStruct((_EP, _PACK), jnp.float32),
            jax.ShapeDtypeStruct((1, 1), jnp.float32),
            jax.ShapeDtypeStruct((1, 1), jnp.float32),
        ],
        compiler_params=pltpu.CompilerParams(
            dimension_semantics=("arbitrary",)),
    )(node_feat, nlab, ef, elab,
      Wn1, bn1.reshape(1, _D), Wn2, bn2.reshape(1, 1),
      We1t, be1t, We2t, be2.reshape(1, 1),
      dn, de)

    return (nlog.reshape(_N), elog.reshape(_E), nsum[0, 0], esum[0, 0])


# speedup vs baseline: 2.1198x; 1.0472x over previous
import jax
import jax.numpy as jnp
from jax import lax
from jax.experimental import pallas as pl
from jax.experimental.pallas import tpu as pltpu

_N, _E, _D, _DE = 10000, 320000, 128, 16
_PACK = _D // _DE
_EP = _E // _PACK
_G = 10
_NBLK = _N // _G
_EBLK = _EP // _G


def _bce(logits, labels):
    return (jnp.maximum(logits, 0.0) - logits * labels
            + jnp.log1p(jnp.exp(-jnp.abs(logits))))


def _fused(nf_ref, nlab_ref, ef_ref, elab_ref,
           wn1_ref, bn1_ref, wn2_ref, bn2_ref,
           we1_ref, be1_ref, we2_ref, be2_ref,
           dn_ref, de_ref,
           nlog_ref, elog_ref, nsum_ref, esum_ref):
    i = pl.program_id(0)

    nh = jnp.maximum(
        jnp.dot(nf_ref[...], wn1_ref[...], preferred_element_type=jnp.float32)
        + bn1_ref[...], 0.0)
    nlogit = (lax.dot_general(wn2_ref[...], nh, (((0,), (1,)), ((), ())),
                              preferred_element_type=jnp.float32)
              + bn2_ref[...])
    nlog_ref[...] = nlogit.reshape(1, 1, _NBLK)

    eh = jnp.maximum(
        jnp.dot(ef_ref[...], we1_ref[...], preferred_element_type=jnp.float32)
        + be1_ref[...], 0.0)
    elogit = (jnp.dot(eh, we2_ref[...], preferred_element_type=jnp.float32)
              + be2_ref[...])
    elog_ref[...] = elogit

    @pl.when(i == 0)
    def _init():
        nsum_ref[...] = jnp.zeros_like(nsum_ref)
        esum_ref[...] = jnp.zeros_like(esum_ref)

    nlab = nlab_ref[...].reshape(1, _NBLK)
    nsum_ref[...] += jnp.sum(_bce(nlogit, nlab)).reshape(1, 1)
    esum_ref[...] += jnp.sum(_bce(elogit, elab_ref[...])).reshape(1, 1)

    @pl.when(i == _G - 1)
    def _norm():
        nsum_ref[...] = nsum_ref[...] / dn_ref[...]
        esum_ref[...] = esum_ref[...] / de_ref[...]


def kernel(node_feat, edge_feat, node_label, edge_label, node_batch,
           edge_batch, Wn1, bn1, Wn2, bn2, We1, be1, We2, be2):
    ef = edge_feat.reshape(_EP, _D)
    elab = edge_label.reshape(_EP, _PACK)
    nlab = node_label.reshape(_G, 1, _NBLK)

    eye = jnp.eye(_PACK, dtype=We1.dtype)
    We1t = jnp.kron(eye, We1)
    be1t = jnp.tile(be1, _PACK).reshape(1, _D)
    We2t = jnp.kron(eye, We2)

    dn = (node_batch[-1].astype(jnp.float32) + 1.0).reshape(1, 1)
    de = (edge_batch[-1].astype(jnp.float32) + 1.0).reshape(1, 1)

    row = lambda i: (i, 0)
    row3 = lambda i: (i, 0, 0)
    fixed = lambda i: (0, 0)
    full = lambda a: pl.BlockSpec(a.shape, fixed)

    nlog, elog, nsum, esum = pl.pallas_call(
        _fused,
        grid=(_G,),
        in_specs=[
            pl.BlockSpec((_NBLK, _D), row),
            pl.BlockSpec((1, 1, _NBLK), row3),
            pl.BlockSpec((_EBLK, _D), row),
            pl.BlockSpec((_EBLK, _PACK), row),
            full(Wn1),
            pl.BlockSpec((1, _D), fixed),
            full(Wn2),
            pl.BlockSpec((1, 1), fixed),
            pl.BlockSpec((_D, _D), fixed),
            pl.BlockSpec((1, _D), fixed),
            pl.BlockSpec((_D, _PACK), fixed),
            pl.BlockSpec((1, 1), fixed),
            pl.BlockSpec((1, 1), fixed),
            pl.BlockSpec((1, 1), fixed),
        ],
        out_specs=[
            pl.BlockSpec((1, 1, _NBLK), row3),
            pl.BlockSpec((_EBLK, _PACK), row),
            pl.BlockSpec((1, 1), fixed),
            pl.BlockSpec((1, 1), fixed),
        ],
        out_shape=[
            jax.ShapeDtypeStruct((_G, 1, _NBLK), jnp.float32),
            jax.ShapeDtypeStruct((_EP, _PACK), jnp.float32),
            jax.ShapeDtypeStruct((1, 1), jnp.float32),
            jax.ShapeDtypeStruct((1, 1), jnp.float32),
        ],
        compiler_params=pltpu.CompilerParams(
            dimension_semantics=("arbitrary",)),
    )(node_feat, nlab, ef, elab,
      Wn1, bn1.reshape(1, _D), Wn2, bn2.reshape(1, 1),
      We1t, be1t, We2t, be2.reshape(1, 1),
      dn, de)

    return (nlog.reshape(_N), elog.reshape(_E), nsum[0, 0], esum[0, 0])


# speedup vs baseline: 7.9883x; 3.7684x over previous
import jax
import jax.numpy as jnp
from jax import lax
from jax.experimental import pallas as pl
from jax.experimental.pallas import tpu as pltpu

_N, _E, _D, _DE = 10000, 320000, 128, 16
_G = 25
_NBLK = _N // _G
_EBLK = _E // _G


def _bce(logits, labels):
    return (jnp.maximum(logits, 0.0) - logits * labels
            + jnp.log1p(jnp.exp(-jnp.abs(logits))))


def _fused(nf_ref, nlab_ref, eft_ref, elab_ref,
           wn1_ref, bn1_ref, wn2t_ref, bn2_ref,
           we1t_ref, be1_ref, we2t_ref, be2_ref,
           dn_ref, de_ref,
           nlog_ref, elog_ref, nsum_ref, esum_ref):
    i = pl.program_id(0)

    nh = jnp.maximum(
        jnp.dot(nf_ref[...], wn1_ref[...], preferred_element_type=jnp.float32)
        + bn1_ref[...], 0.0)
    nlogit = (lax.dot_general(wn2t_ref[...], nh, (((1,), (1,)), ((), ())),
                              preferred_element_type=jnp.float32)
              + bn2_ref[...])
    nlog_ref[...] = nlogit.reshape(1, 1, _NBLK)

    eh = jnp.maximum(
        jnp.dot(we1t_ref[...], eft_ref[...], preferred_element_type=jnp.float32)
        + be1_ref[...], 0.0)
    elogit = (jnp.dot(we2t_ref[...], eh, preferred_element_type=jnp.float32)
              + be2_ref[...])
    elog_ref[...] = elogit.reshape(1, 1, _EBLK)

    @pl.when(i == 0)
    def _init():
        nsum_ref[...] = jnp.zeros_like(nsum_ref)
        esum_ref[...] = jnp.zeros_like(esum_ref)

    nlab = nlab_ref[...].reshape(1, _NBLK)
    elab = elab_ref[...].reshape(1, _EBLK)
    nsum_ref[...] += jnp.sum(_bce(nlogit, nlab)).reshape(1, 1)
    esum_ref[...] += jnp.sum(_bce(elogit, elab)).reshape(1, 1)

    @pl.when(i == _G - 1)
    def _norm():
        nsum_ref[...] = nsum_ref[...] / dn_ref[...]
        esum_ref[...] = esum_ref[...] / de_ref[...]


def kernel(node_feat, edge_feat, node_label, edge_label, node_batch,
           edge_batch, Wn1, bn1, Wn2, bn2, We1, be1, We2, be2):
    eft = edge_feat.T
    elab = edge_label.reshape(_G, 1, _EBLK)
    nlab = node_label.reshape(_G, 1, _NBLK)

    dn = (node_batch[-1].astype(jnp.float32) + 1.0).reshape(1, 1)
    de = (edge_batch[-1].astype(jnp.float32) + 1.0).reshape(1, 1)

    row3 = lambda i: (i, 0, 0)
    col = lambda i: (0, i)
    fixed = lambda i: (0, 0)
    full = lambda a: pl.BlockSpec(a.shape, fixed)

    nlog, elog, nsum, esum = pl.pallas_call(
        _fused,
        grid=(_G,),
        in_specs=[
            pl.BlockSpec((_NBLK, _D), lambda i: (i, 0)),
            pl.BlockSpec((1, 1, _NBLK), row3),
            pl.BlockSpec((_DE, _EBLK), col),
            pl.BlockSpec((1, 1, _EBLK), row3),
            full(Wn1),
            pl.BlockSpec((1, _D), fixed),
            pl.BlockSpec((1, _D), fixed),
            pl.BlockSpec((1, 1), fixed),
            pl.BlockSpec((_DE, _DE), fixed),
            pl.BlockSpec((_DE, 1), fixed),
            pl.BlockSpec((1, _DE), fixed),
            pl.BlockSpec((1, 1), fixed),
            pl.BlockSpec((1, 1), fixed),
            pl.BlockSpec((1, 1), fixed),
        ],
        out_specs=[
            pl.BlockSpec((1, 1, _NBLK), row3),
            pl.BlockSpec((1, 1, _EBLK), row3),
            pl.BlockSpec((1, 1), fixed),
            pl.BlockSpec((1, 1), fixed),
        ],
        out_shape=[
            jax.ShapeDtypeStruct((_G, 1, _NBLK), jnp.float32),
            jax.ShapeDtypeStruct((_G, 1, _EBLK), jnp.float32),
            jax.ShapeDtypeStruct((1, 1), jnp.float32),
            jax.ShapeDtypeStruct((1, 1), jnp.float32),
        ],
        compiler_params=pltpu.CompilerParams(
            dimension_semantics=("arbitrary",)),
    )(node_feat, nlab, eft, elab,
      Wn1, bn1.reshape(1, _D), Wn2.T, bn2.reshape(1, 1),
      We1.T, be1.reshape(_DE, 1), We2.T, be2.reshape(1, 1),
      dn, de)

    return (nlog.reshape(_N), elog.reshape(_E), nsum[0, 0], esum[0, 0])
